# Initial kernel scaffold; baseline (speedup 1.0000x reference)
import jax
import jax.numpy as jnp
from jax.experimental import pallas as pl


def kernel(time, pe):
    raise NotImplementedError("write your pallas kernel here")


# speedup vs baseline: 11.8432x; 11.8432x over previous
import functools

import jax
import jax.numpy as jnp
from jax import lax
from jax.experimental import pallas as pl
from jax.experimental.pallas import tpu as pltpu
from jax.experimental.pallas import tpu_sc as plsc

D = 128
ROWS = 367
GROUP = 128
NBUF = 5


def _pe_lookup(idx_flat, pe, *, n_idx):
    info = plsc.get_sparse_core_info()
    nc, ns = info.num_cores, info.num_subcores
    nw = nc * ns
    gpw = n_idx // (nw * GROUP)
    n_outer = gpw // NBUF
    assert n_idx == nw * gpw * GROUP and gpw % NBUF == 0

    mesh = plsc.VectorSubcoreMesh(core_axis_name="c", subcore_axis_name="s")

    @functools.partial(
        pl.kernel,
        mesh=mesh,
        out_type=jax.ShapeDtypeStruct((n_idx, D), jnp.float32),
        scratch_types=[
            pltpu.VMEM_SHARED((ROWS, D), jnp.float32),
            pltpu.VMEM((gpw * GROUP,), jnp.int32),
        ]
        + [pltpu.VMEM((GROUP, D), jnp.float32) for _ in range(NBUF)]
        + [pltpu.SemaphoreType.DMA for _ in range(2 * NBUF)],
    )
    def k(idx_hbm, pe_hbm, out_hbm, tab_sh, idx_v, *rest):
        rows = rest[:NBUF]
        gsem = rest[NBUF:2 * NBUF]
        ssem = rest[2 * NBUF:]

        c = lax.axis_index("c")
        s = lax.axis_index("s")
        wid = s * nc + c

        @pl.when(s == 0)
        def _():
            pltpu.sync_copy(pe_hbm, tab_sh)

        plsc.subcore_barrier()

        pltpu.sync_copy(idx_hbm.at[pl.ds(wid * gpw * GROUP, gpw * GROUP)], idx_v)

        for b in range(NBUF):
            pltpu.async_copy(
                tab_sh.at[idx_v.at[pl.ds(b * GROUP, GROUP)]], rows[b], gsem[b])

        def outer(o, carry):
            for b in range(NBUF):
                g = o * NBUF + b
                pltpu.make_async_copy(
                    pe_hbm.at[pl.ds(0, GROUP)], rows[b], gsem[b]).wait()
                row_base = (wid * gpw + g) * GROUP
                pltpu.async_copy(
                    rows[b], out_hbm.at[pl.ds(row_base, GROUP)], ssem[b])
            for b in range(NBUF):
                pltpu.make_async_copy(
                    rows[b], out_hbm.at[pl.ds(0, GROUP)], ssem[b]).wait()

                @pl.when(o < n_outer - 1)
                def _():
                    g_next = (o + 1) * NBUF + b
                    pltpu.async_copy(
                        tab_sh.at[idx_v.at[pl.ds(g_next * GROUP, GROUP)]],
                        rows[b], gsem[b])
            return carry

        lax.fori_loop(0, n_outer, outer, 0)

    return k(idx_flat, pe)


def kernel(time, pe):
    b, t = time.shape
    n_idx = b * t
    idx_flat = jnp.reshape(time.astype(jnp.int32), (n_idx,))
    out = _pe_lookup(idx_flat, pe, n_idx=n_idx)
    return jnp.reshape(out, (b, t, D))


# speedup vs baseline: 11.8585x; 1.0013x over previous
import functools

import jax
import jax.numpy as jnp
from jax import lax
from jax.experimental import pallas as pl
from jax.experimental.pallas import tpu as pltpu
from jax.experimental.pallas import tpu_sc as plsc

D = 128
ROWS = 367
GMAX = 128
NBUF = 4


def kernel(time, pe):
    bsz, t = time.shape
    info = plsc.get_sparse_core_info()
    nc, ns = info.num_cores, info.num_subcores
    nw = nc * ns
    rpw = bsz // nw
    n_outer = rpw // NBUF
    assert bsz == nw * rpw and rpw % NBUF == 0 and t <= 2 * GMAX

    mesh = plsc.VectorSubcoreMesh(core_axis_name="c", subcore_axis_name="s")

    @functools.partial(
        pl.kernel,
        mesh=mesh,
        out_type=jax.ShapeDtypeStruct((bsz, t, D), jnp.float32),
        scratch_types=[
            pltpu.VMEM_SHARED((ROWS, D), jnp.float32),
            pltpu.VMEM((rpw, t), jnp.int32),
        ]
        + [pltpu.VMEM((t, D), jnp.float32) for _ in range(NBUF)]
        + [pltpu.SemaphoreType.DMA for _ in range(2 * NBUF)],
    )
    def k(idx_hbm, pe_hbm, out_hbm, tab_sh, idx_v, *rest):
        rows = rest[:NBUF]
        gsem = rest[NBUF:2 * NBUF]
        ssem = rest[2 * NBUF:]

        c = lax.axis_index("c")
        s = lax.axis_index("s")
        wid = s * nc + c

        @pl.when(s == 0)
        def _():
            pltpu.sync_copy(pe_hbm, tab_sh)

        plsc.subcore_barrier()

        pltpu.sync_copy(idx_hbm.at[pl.ds(wid * rpw, rpw)], idx_v)

        def gather_row(r, b):
            pltpu.async_copy(
                tab_sh.at[idx_v.at[r, pl.ds(0, GMAX)]],
                rows[b].at[pl.ds(0, GMAX)], gsem[b])
            pltpu.async_copy(
                tab_sh.at[idx_v.at[r, pl.ds(GMAX, t - GMAX)]],
                rows[b].at[pl.ds(GMAX, t - GMAX)], gsem[b])

        for b in range(NBUF):
            gather_row(b, b)

        def outer(o, carry):
            for b in range(NBUF):
                r = o * NBUF + b
                pltpu.make_async_copy(
                    pe_hbm.at[pl.ds(0, t)], rows[b], gsem[b]).wait()
                pltpu.async_copy(rows[b], out_hbm.at[wid * rpw + r], ssem[b])
            for b in range(NBUF):
                pltpu.make_async_copy(rows[b], out_hbm.at[0], ssem[b]).wait()

                @pl.when(o < n_outer - 1)
                def _():
                    gather_row((o + 1) * NBUF + b, b)
            return carry

        lax.fori_loop(0, n_outer, outer, 0)

    return k(time.astype(jnp.int32), pe)
